# Initial kernel scaffold; baseline (speedup 1.0000x reference)
import jax
import jax.numpy as jnp
from jax.experimental import pallas as pl


def kernel(t, idx, v):
    raise NotImplementedError("write your pallas kernel here")


# speedup vs baseline: 6.4019x; 6.4019x over previous
import functools

import jax
import jax.numpy as jnp
from jax.experimental import pallas as pl

_D0, _N, _D2, _C = 2, 1024, 16, 1024
_R = 64


def _diag_set_kernel(idx_ref, v_ref, t_ref, o_ref):
    i = pl.program_id(0)
    tb = t_ref[...]
    rows = i * _R + jax.lax.broadcasted_iota(jnp.int32, (_R, 1), 0)
    eq = idx_ref[...] == rows
    member = eq.any(axis=1)
    vsel = jnp.dot(eq.astype(jnp.float32), v_ref[...],
                   preferred_element_type=jnp.float32)
    vsel = vsel.reshape(_R, _D0, _D2).transpose(1, 0, 2)
    col = jax.lax.broadcasted_iota(jnp.int32, (_D0, _R, _D2, _C), 3)
    rowg = jax.lax.broadcasted_iota(jnp.int32, (_D0, _R, _D2, _C), 1) + i * _R
    mask = (col == rowg) & member[None, :, None, None]
    o_ref[...] = jnp.where(mask, vsel[..., None], tb)


@functools.partial(jax.jit, static_argnames=())
def kernel(t, idx, v):
    idx2 = idx.reshape(1, _N).astype(jnp.int32)
    v2 = v.reshape(_N, _D0 * _D2)
    grid = (_N // _R,)
    return pl.pallas_call(
        _diag_set_kernel,
        grid=grid,
        in_specs=[
            pl.BlockSpec((1, _N), lambda i: (0, 0)),
            pl.BlockSpec((_N, _D0 * _D2), lambda i: (0, 0)),
            pl.BlockSpec((_D0, _R, _D2, _C), lambda i: (0, i, 0, 0)),
        ],
        out_specs=pl.BlockSpec((_D0, _R, _D2, _C), lambda i: (0, i, 0, 0)),
        out_shape=jax.ShapeDtypeStruct(t.shape, t.dtype),
    )(idx2, v2, t)
